# baseline (speedup 1.0000x reference)
import jax
import jax.numpy as jnp
from jax import lax
from jax.experimental import pallas as pl
from jax.experimental.pallas import tpu as pltpu
from jax.experimental.pallas import tpu_sc as plsc

B = 128
V = 100000
RB = 8
NRB = B // RB
CW = 1280
NFULL = V // CW
TAILW = V - NFULL * CW
TCCH = 68
TCC = TCCH * CW
SCFULL = NFULL - TCCH
CPH = SCFULL // 2
NV = CW // 16
TCW = 10880
TCSTEPS = TCC // TCW


def _lanes():
    return lax.broadcasted_iota(jnp.int32, (16,), 0)


def _chunk_update(buf, width, ms):
    nv = width // 16

    def _max_body(i, carry):
        return tuple(
            jnp.maximum(carry[r], buf[r, pl.ds(i * 16, 16)])
            for r in range(RB))

    m_new = lax.fori_loop(0, nv, _max_body,
                          tuple(m for m, _ in ms), unroll=4)
    s_scaled = tuple(
        s * jnp.exp(m - m_new[r]) for r, (m, s) in enumerate(ms))

    def _sum_body(i, carry):
        return tuple(
            carry[r] + jnp.exp(buf[r, pl.ds(i * 16, 16)] - m_new[r])
            for r in range(RB))

    s_new = lax.fori_loop(0, nv, _sum_body, s_scaled, unroll=4)
    return [(m_new[r], s_new[r]) for r in range(RB)]


def _sc_body(pred_hbm, act_hbm, ms_hbm, g_hbm,
             buf0, buf1, tbuf, abuf, gbuf, stage, gstage,
             sem0, sem1, tsem, asem, gsem, osem):
    rb = lax.axis_index("s")
    h = lax.axis_index("c")
    row0 = rb * RB

    def start(c_idx, buf, sem):
        col = (TCCH + 2 * c_idx + h) * CW
        return pltpu.async_copy(
            pred_hbm.at[pl.ds(row0, RB), pl.ds(col, CW)], buf, sem)

    start(0, buf0, sem0)
    start(1, buf1, sem1)

    neg_inf = jnp.full((16,), -jnp.inf, dtype=jnp.float32)
    zero = jnp.zeros((16,), dtype=jnp.float32)
    ms = [(neg_inf, zero) for _ in range(RB)]

    def _flatten(ms):
        return sum(ms, ())

    def _unflatten(t):
        return [(t[2 * r], t[2 * r + 1]) for r in range(RB)]

    def pair_body(cc, carry):
        ms = _unflatten(carry)
        pltpu.make_async_copy(
            pred_hbm.at[pl.ds(row0, RB), pl.ds(0, CW)], buf0, sem0).wait()
        ms = _chunk_update(buf0, CW, ms)

        @pl.when(2 * cc + 2 < CPH)
        def _():
            start(2 * cc + 2, buf0, sem0)

        pltpu.make_async_copy(
            pred_hbm.at[pl.ds(row0, RB), pl.ds(0, CW)], buf1, sem1).wait()
        ms = _chunk_update(buf1, CW, ms)

        @pl.when(2 * cc + 3 < CPH)
        def _():
            start(2 * cc + 3, buf1, sem1)

        return _flatten(ms)

    carry = lax.fori_loop(0, CPH // 2, pair_body, _flatten(ms))
    ms = _unflatten(carry)
    pltpu.make_async_copy(
        pred_hbm.at[pl.ds(row0, RB), pl.ds(0, CW)], buf0, sem0).wait()
    ms = _chunk_update(buf0, CW, ms)

    @pl.when(h == 1)
    def _():
        pltpu.async_copy(
            pred_hbm.at[pl.ds(row0, RB), pl.ds(NFULL * CW, TAILW)],
            tbuf, tsem).wait()

    tms = _chunk_update(tbuf, TAILW, ms)

    def _sel(pair_t, pair_f):
        pred = h == 1
        return [(jnp.where(pred, a1, a0), jnp.where(pred, b1, b0))
                for (a1, b1), (a0, b0) in zip(pair_t, pair_f)]

    ms = _sel(tms, ms)

    out_vec = zero
    for r in range(RB):
        m_vec, s_vec = ms[r]
        m_row = jnp.max(m_vec)
        s_row = jnp.sum(s_vec * jnp.exp(m_vec - jnp.full((16,), m_row)))
        out_vec = out_vec + jnp.where(_lanes() == r, m_row, 0.0)
        out_vec = out_vec + jnp.where(_lanes() == RB + r, s_row, 0.0)
    stage[...] = out_vec
    pltpu.async_copy(stage, ms_hbm.at[h, rb], osem).wait()

    @pl.when(h == 0)
    def _():
        pltpu.async_copy(act_hbm.at[pl.ds(rb * RB, 16)], abuf, asem).wait()
        avec = abuf[...]
        gv = zero
        for r in range(RB):
            a = jnp.sum(jnp.where(_lanes() == r, avec, 0))
            base = pl.multiple_of(a - lax.rem(a, 128), 128)
            in_tail = base >= NFULL * CW

            @pl.when(jnp.logical_not(in_tail))
            def _():
                pltpu.async_copy(
                    pred_hbm.at[pl.ds(row0, RB), pl.ds(base, 128)],
                    gbuf, gsem).wait()

            @pl.when(in_tail)
            def _():
                pltpu.async_copy(
                    pred_hbm.at[pl.ds(row0, RB), pl.ds(NFULL * CW, TAILW)],
                    tbuf, tsem).wait()

            off_g = a - base
            off_t = a - NFULL * CW
            acc = zero
            for v in range(8):
                acc = acc + jnp.where(_lanes() == off_g - v * 16,
                                      gbuf[r, pl.ds(v * 16, 16)], 0.0)
            acc_t = zero
            for v in range(TAILW // 16):
                acc_t = acc_t + jnp.where(_lanes() == off_t - v * 16,
                                          tbuf[r, pl.ds(v * 16, 16)], 0.0)
            val = jnp.where(in_tail, jnp.sum(acc_t), jnp.sum(acc))
            gv = gv + jnp.where(_lanes() == r, val, 0.0)
        gstage[...] = gv
        pltpu.async_copy(gstage, g_hbm.at[rb], gsem).wait()


def _tc_partial_body(x_ref, m_out, s_out, m_acc, s_acc):
    k = pl.program_id(0)

    @pl.when(k == 0)
    def _():
        m_acc[...] = jnp.full_like(m_acc, -jnp.inf)
        s_acc[...] = jnp.zeros_like(s_acc)

    x = x_ref[...]
    bm = jnp.max(x, axis=1, keepdims=True)
    m_old = m_acc[...]
    m_new = jnp.maximum(m_old, bm)
    s_acc[...] = s_acc[...] * jnp.exp(m_old - m_new) + jnp.sum(
        jnp.exp(x - m_new), axis=1, keepdims=True)
    m_acc[...] = m_new

    @pl.when(k == TCSTEPS - 1)
    def _():
        m_out[...] = m_acc[...]
        s_out[...] = s_acc[...]


def _combine_body(ms_ref, g_ref, mtc_ref, stc_ref, out_ref):
    o0 = ms_ref[0]
    o1 = ms_ref[1]
    m0, s0 = o0[:, 0:RB], o0[:, RB:2 * RB]
    m1, s1 = o1[:, 0:RB], o1[:, RB:2 * RB]
    m2, s2 = mtc_ref[...], stc_ref[...]
    m = jnp.maximum(jnp.maximum(m0, m1), m2)
    s = (s0 * jnp.exp(m0 - m) + s1 * jnp.exp(m1 - m)
         + s2 * jnp.exp(m2 - m))
    out_ref[...] = g_ref[:, 0:RB] - m - jnp.log(s)


@jax.jit
def kernel(prediction, action):
    act = jnp.pad(action.astype(jnp.int32), (0, 128))
    mesh = plsc.VectorSubcoreMesh(core_axis_name="c", subcore_axis_name="s")
    sc = pl.kernel(
        _sc_body,
        mesh=mesh,
        compiler_params=pltpu.CompilerParams(needs_layout_passes=False),
        out_type=(
            jax.ShapeDtypeStruct((2, NRB, 16), jnp.float32),
            jax.ShapeDtypeStruct((NRB, 16), jnp.float32),
        ),
        scratch_types=[
            pltpu.VMEM((RB, CW), jnp.float32),
            pltpu.VMEM((RB, CW), jnp.float32),
            pltpu.VMEM((RB, TAILW), jnp.float32),
            pltpu.VMEM((16,), jnp.int32),
            pltpu.VMEM((RB, 128), jnp.float32),
            pltpu.VMEM((16,), jnp.float32),
            pltpu.VMEM((16,), jnp.float32),
            pltpu.SemaphoreType.DMA,
            pltpu.SemaphoreType.DMA,
            pltpu.SemaphoreType.DMA,
            pltpu.SemaphoreType.DMA,
            pltpu.SemaphoreType.DMA,
            pltpu.SemaphoreType.DMA,
        ],
    )
    ms, g = sc(prediction, act)
    mtc, stc = pl.pallas_call(
        _tc_partial_body,
        grid=(TCSTEPS,),
        in_specs=[pl.BlockSpec((B, TCW), lambda k: (0, k))],
        out_specs=(
            pl.BlockSpec((B, 1), lambda k: (0, 0)),
            pl.BlockSpec((B, 1), lambda k: (0, 0)),
        ),
        out_shape=(
            jax.ShapeDtypeStruct((B, 1), jnp.float32),
            jax.ShapeDtypeStruct((B, 1), jnp.float32),
        ),
        scratch_shapes=[
            pltpu.VMEM((B, 1), jnp.float32),
            pltpu.VMEM((B, 1), jnp.float32),
        ],
    )(prediction)
    out = pl.pallas_call(
        _combine_body,
        in_specs=[
            pl.BlockSpec((2, NRB, 16), lambda: (0, 0, 0)),
            pl.BlockSpec((NRB, 16), lambda: (0, 0)),
            pl.BlockSpec((NRB, RB), lambda: (0, 0)),
            pl.BlockSpec((NRB, RB), lambda: (0, 0)),
        ],
        out_specs=pl.BlockSpec((NRB, RB), lambda: (0, 0)),
        out_shape=jax.ShapeDtypeStruct((NRB, RB), jnp.float32),
    )(ms, g, mtc.reshape(NRB, RB), stc.reshape(NRB, RB))
    return out.reshape(B)


# speedup vs baseline: 1.0039x; 1.0039x over previous
import jax
import jax.numpy as jnp
from jax import lax
from jax.experimental import pallas as pl
from jax.experimental.pallas import tpu as pltpu
from jax.experimental.pallas import tpu_sc as plsc

B = 128
V = 100000
RB = 8
NRB = B // RB
CW = 1280
NFULL = V // CW
TAILW = V - NFULL * CW
TCCH = 64
TCC = TCCH * CW
SCFULL = NFULL - TCCH
CPH = SCFULL // 2
NV = CW // 16
TCW = 10240
TCSTEPS = TCC // TCW


def _lanes():
    return lax.broadcasted_iota(jnp.int32, (16,), 0)


def _chunk_update(buf, width, ms):
    nv = width // 16

    def _max_body(i, carry):
        return tuple(
            jnp.maximum(carry[r], buf[r, pl.ds(i * 16, 16)])
            for r in range(RB))

    m_new = lax.fori_loop(0, nv, _max_body,
                          tuple(m for m, _ in ms), unroll=4)
    s_scaled = tuple(
        s * jnp.exp(m - m_new[r]) for r, (m, s) in enumerate(ms))

    def _sum_body(i, carry):
        return tuple(
            carry[r] + jnp.exp(buf[r, pl.ds(i * 16, 16)] - m_new[r])
            for r in range(RB))

    s_new = lax.fori_loop(0, nv, _sum_body, s_scaled, unroll=4)
    return [(m_new[r], s_new[r]) for r in range(RB)]


def _sc_body(pred_hbm, act_hbm, ms_hbm, g_hbm,
             buf0, buf1, tbuf, abuf, gbuf, stage, gstage,
             sem0, sem1, tsem, asem, gsem, osem):
    rb = lax.axis_index("s")
    h = lax.axis_index("c")
    row0 = rb * RB

    def start(c_idx, buf, sem):
        col = (TCCH + 2 * c_idx + h) * CW
        return pltpu.async_copy(
            pred_hbm.at[pl.ds(row0, RB), pl.ds(col, CW)], buf, sem)

    start(0, buf0, sem0)
    start(1, buf1, sem1)

    neg_inf = jnp.full((16,), -jnp.inf, dtype=jnp.float32)
    zero = jnp.zeros((16,), dtype=jnp.float32)
    ms = [(neg_inf, zero) for _ in range(RB)]

    def _flatten(ms):
        return sum(ms, ())

    def _unflatten(t):
        return [(t[2 * r], t[2 * r + 1]) for r in range(RB)]

    def pair_body(cc, carry):
        ms = _unflatten(carry)
        pltpu.make_async_copy(
            pred_hbm.at[pl.ds(row0, RB), pl.ds(0, CW)], buf0, sem0).wait()
        ms = _chunk_update(buf0, CW, ms)

        @pl.when(2 * cc + 2 < CPH)
        def _():
            start(2 * cc + 2, buf0, sem0)

        pltpu.make_async_copy(
            pred_hbm.at[pl.ds(row0, RB), pl.ds(0, CW)], buf1, sem1).wait()
        ms = _chunk_update(buf1, CW, ms)

        @pl.when(2 * cc + 3 < CPH)
        def _():
            start(2 * cc + 3, buf1, sem1)

        return _flatten(ms)

    carry = lax.fori_loop(0, CPH // 2, pair_body, _flatten(ms))
    ms = _unflatten(carry)
    pltpu.make_async_copy(
        pred_hbm.at[pl.ds(row0, RB), pl.ds(0, CW)], buf0, sem0).wait()
    ms = _chunk_update(buf0, CW, ms)

    @pl.when(h == 1)
    def _():
        pltpu.async_copy(
            pred_hbm.at[pl.ds(row0, RB), pl.ds(NFULL * CW, TAILW)],
            tbuf, tsem).wait()

    tms = _chunk_update(tbuf, TAILW, ms)

    def _sel(pair_t, pair_f):
        pred = h == 1
        return [(jnp.where(pred, a1, a0), jnp.where(pred, b1, b0))
                for (a1, b1), (a0, b0) in zip(pair_t, pair_f)]

    ms = _sel(tms, ms)

    out_vec = zero
    for r in range(RB):
        m_vec, s_vec = ms[r]
        m_row = jnp.max(m_vec)
        s_row = jnp.sum(s_vec * jnp.exp(m_vec - jnp.full((16,), m_row)))
        out_vec = out_vec + jnp.where(_lanes() == r, m_row, 0.0)
        out_vec = out_vec + jnp.where(_lanes() == RB + r, s_row, 0.0)
    stage[...] = out_vec
    pltpu.async_copy(stage, ms_hbm.at[h, rb], osem).wait()

    @pl.when(h == 0)
    def _():
        pltpu.async_copy(act_hbm.at[pl.ds(rb * RB, 16)], abuf, asem).wait()
        avec = abuf[...]
        gv = zero
        for r in range(RB):
            a = jnp.sum(jnp.where(_lanes() == r, avec, 0))
            base = pl.multiple_of(a - lax.rem(a, 128), 128)
            in_tail = base >= NFULL * CW

            @pl.when(jnp.logical_not(in_tail))
            def _():
                pltpu.async_copy(
                    pred_hbm.at[pl.ds(row0, RB), pl.ds(base, 128)],
                    gbuf, gsem).wait()

            @pl.when(in_tail)
            def _():
                pltpu.async_copy(
                    pred_hbm.at[pl.ds(row0, RB), pl.ds(NFULL * CW, TAILW)],
                    tbuf, tsem).wait()

            off_g = a - base
            off_t = a - NFULL * CW
            acc = zero
            for v in range(8):
                acc = acc + jnp.where(_lanes() == off_g - v * 16,
                                      gbuf[r, pl.ds(v * 16, 16)], 0.0)
            acc_t = zero
            for v in range(TAILW // 16):
                acc_t = acc_t + jnp.where(_lanes() == off_t - v * 16,
                                          tbuf[r, pl.ds(v * 16, 16)], 0.0)
            val = jnp.where(in_tail, jnp.sum(acc_t), jnp.sum(acc))
            gv = gv + jnp.where(_lanes() == r, val, 0.0)
        gstage[...] = gv
        pltpu.async_copy(gstage, g_hbm.at[rb], gsem).wait()


def _tc_partial_body(x_ref, m_out, s_out, m_acc, s_acc):
    k = pl.program_id(0)

    @pl.when(k == 0)
    def _():
        m_acc[...] = jnp.full_like(m_acc, -jnp.inf)
        s_acc[...] = jnp.zeros_like(s_acc)

    x = x_ref[...]
    bm = jnp.max(x, axis=1, keepdims=True)
    m_old = m_acc[...]
    m_new = jnp.maximum(m_old, bm)
    s_acc[...] = s_acc[...] * jnp.exp(m_old - m_new) + jnp.sum(
        jnp.exp(x - m_new), axis=1, keepdims=True)
    m_acc[...] = m_new

    @pl.when(k == TCSTEPS - 1)
    def _():
        m_out[...] = m_acc[...]
        s_out[...] = s_acc[...]


def _combine_body(ms_ref, g_ref, mtc_ref, stc_ref, out_ref):
    o0 = ms_ref[0]
    o1 = ms_ref[1]
    m0, s0 = o0[:, 0:RB], o0[:, RB:2 * RB]
    m1, s1 = o1[:, 0:RB], o1[:, RB:2 * RB]
    m2, s2 = mtc_ref[...], stc_ref[...]
    m = jnp.maximum(jnp.maximum(m0, m1), m2)
    s = (s0 * jnp.exp(m0 - m) + s1 * jnp.exp(m1 - m)
         + s2 * jnp.exp(m2 - m))
    out_ref[...] = g_ref[:, 0:RB] - m - jnp.log(s)


@jax.jit
def kernel(prediction, action):
    act = jnp.pad(action.astype(jnp.int32), (0, 128))
    mesh = plsc.VectorSubcoreMesh(core_axis_name="c", subcore_axis_name="s")
    sc = pl.kernel(
        _sc_body,
        mesh=mesh,
        compiler_params=pltpu.CompilerParams(needs_layout_passes=False),
        out_type=(
            jax.ShapeDtypeStruct((2, NRB, 16), jnp.float32),
            jax.ShapeDtypeStruct((NRB, 16), jnp.float32),
        ),
        scratch_types=[
            pltpu.VMEM((RB, CW), jnp.float32),
            pltpu.VMEM((RB, CW), jnp.float32),
            pltpu.VMEM((RB, TAILW), jnp.float32),
            pltpu.VMEM((16,), jnp.int32),
            pltpu.VMEM((RB, 128), jnp.float32),
            pltpu.VMEM((16,), jnp.float32),
            pltpu.VMEM((16,), jnp.float32),
            pltpu.SemaphoreType.DMA,
            pltpu.SemaphoreType.DMA,
            pltpu.SemaphoreType.DMA,
            pltpu.SemaphoreType.DMA,
            pltpu.SemaphoreType.DMA,
            pltpu.SemaphoreType.DMA,
        ],
    )
    ms, g = sc(prediction, act)
    mtc, stc = pl.pallas_call(
        _tc_partial_body,
        grid=(TCSTEPS,),
        in_specs=[pl.BlockSpec((B, TCW), lambda k: (0, k))],
        out_specs=(
            pl.BlockSpec((B, 1), lambda k: (0, 0)),
            pl.BlockSpec((B, 1), lambda k: (0, 0)),
        ),
        out_shape=(
            jax.ShapeDtypeStruct((B, 1), jnp.float32),
            jax.ShapeDtypeStruct((B, 1), jnp.float32),
        ),
        scratch_shapes=[
            pltpu.VMEM((B, 1), jnp.float32),
            pltpu.VMEM((B, 1), jnp.float32),
        ],
    )(prediction)
    out = pl.pallas_call(
        _combine_body,
        in_specs=[
            pl.BlockSpec((2, NRB, 16), lambda: (0, 0, 0)),
            pl.BlockSpec((NRB, 16), lambda: (0, 0)),
            pl.BlockSpec((NRB, RB), lambda: (0, 0)),
            pl.BlockSpec((NRB, RB), lambda: (0, 0)),
        ],
        out_specs=pl.BlockSpec((NRB, RB), lambda: (0, 0)),
        out_shape=jax.ShapeDtypeStruct((NRB, RB), jnp.float32),
    )(ms, g, mtc.reshape(NRB, RB), stc.reshape(NRB, RB))
    return out.reshape(B)
